# baseline (speedup 1.0000x reference)
import functools

import jax
import jax.numpy as jnp
from jax import lax
from jax.experimental import pallas as pl
from jax.experimental.pallas import tpu as pltpu
from jax.experimental.pallas import tpu_sc as plsc

N = 10000
E = 320000
D = 128
DH = D // 2

NC = 2
NS = 16
EPT = E // NS
CH = 80
NCHUNK = EPT // CH
STRIPE_A = 632
PIECES_A = (120, 120, 120, 120, 120, 32)
PIECES_B = (120, 120, 120, 120, 40)

ROWS_BLK = 1000
GRID = N // ROWS_BLK

_mesh = plsc.VectorSubcoreMesh(core_axis_name="c", subcore_axis_name="s")


@functools.partial(
    pl.kernel,
    out_type=jax.ShapeDtypeStruct((NC, N, DH), jnp.float32),
    mesh=_mesh,
    compiler_params=pltpu.CompilerParams(needs_layout_passes=False,
                                         use_tc_tiling_on_sc=False),
    scratch_types=[
        pltpu.VMEM((NCHUNK, CH), jnp.int32),
        pltpu.VMEM((NCHUNK, CH), jnp.int32),
        pltpu.VMEM((N,), jnp.float32),
        pltpu.VMEM((N,), jnp.float32),
        pltpu.VMEM((CH, DH), jnp.float32),
        pltpu.VMEM((CH, DH), jnp.float32),
        pltpu.VMEM((CH, DH), jnp.float32),
        pltpu.VMEM((CH, DH), jnp.float32),
        pltpu.VMEM((128,), jnp.float32),
        pltpu.VMEM((128,), jnp.float32),
        pltpu.VMEM((120, DH), jnp.float32),
        pltpu.VMEM((120,), jnp.float32),
        pltpu.VMEM_SHARED((N, DH), jnp.float32),
        pltpu.VMEM_SHARED((N,), jnp.float32),
        pltpu.SemaphoreType.DMA,
        pltpu.SemaphoreType.DMA,
        pltpu.SemaphoreType.DMA,
    ],
)
def _sc_edge_layer(h_hbm, src_hbm, dst_hbm, es_hbm, ed_hbm, acc_out,
                   src_v, dst_v, es_v, ed_v, rows_in0, rows_in1, rows_sc0,
                   rows_sc1, ee_buf, ee_buf1, zbuf, den_v, acc_sp, den_sp,
                   sem, sem_r, sem_d):
    cid = lax.axis_index("c")
    sid = lax.axis_index("s")
    base = sid * STRIPE_A

    @pl.loop(0, 120)
    def _zb(i):
        for g in range(DH // 16):
            zbuf[i, pl.ds(g * 16, 16)] = jnp.zeros((16,), jnp.float32)

    def _stripe_zero(pieces):
        off = 0
        for p in pieces:
            pltpu.sync_copy(zbuf.at[pl.ds(0, p)],
                            acc_sp.at[pl.ds(base + off, p)])
            off += p

    @pl.when(sid < NS - 1)
    def _za():
        _stripe_zero(PIECES_A)

    @pl.when(sid == NS - 1)
    def _zbb():
        _stripe_zero(PIECES_B)

    @pl.when(sid == 0)
    def _zden():
        @pl.loop(0, N // 128)
        def _z(i):
            for g in range(8):
                es_v[pl.ds(i * 128 + g * 16, 16)] = jnp.zeros(
                    (16,), jnp.float32)
        pltpu.sync_copy(es_v, den_sp)

    pltpu.sync_copy(src_hbm.at[sid], src_v)
    pltpu.sync_copy(dst_hbm.at[sid], dst_v)
    pltpu.sync_copy(es_hbm, es_v)
    pltpu.sync_copy(ed_hbm, ed_v)

    def _vmax(ref):
        @pl.loop(0, N // 128, init_carry=jnp.full((16,), -jnp.inf, jnp.float32))
        def m(i, carry):
            for g in range(8):
                carry = jnp.maximum(carry, ref[pl.ds(i * 128 + g * 16, 16)])
            return carry
        out = m
        for sh in (8, 4, 2, 1):
            ee_buf[pl.ds(0, 16)] = out
            idx = lax.iota(jnp.int32, 16) ^ sh
            out = jnp.maximum(out, plsc.load_gather(ee_buf, [idx]))
        return out

    c0 = _vmax(es_v) + _vmax(ed_v)
    c = jnp.where(c0 > 0, c0, 0.2 * c0)

    plsc.subcore_barrier()

    def _gather(j, buf):
        pltpu.async_copy(h_hbm.at[cid].at[src_v.at[j]], buf, sem)

    def _scat_wait(j, rows_sc, ee):
        pltpu.make_async_copy(rows_sc, acc_sp.at[dst_v.at[j]], sem_r).wait()
        pltpu.make_async_copy(ee.at[pl.ds(0, CH)], den_sp.at[dst_v.at[j]],
                              sem_d).wait()

    def _process(j, rows_in, rows_sc, ee_st):
        @pl.when(j >= 2)
        def _wd():
            pltpu.make_async_copy(ee_st.at[pl.ds(0, CH)],
                                  den_sp.at[dst_v.at[j]], sem_d).wait()

        ees = []
        for g in range(CH // 16):
            sidx = src_v[j, pl.ds(g * 16, 16)]
            didx = dst_v[j, pl.ds(g * 16, 16)]
            el = plsc.load_gather(es_v, [sidx]) + plsc.load_gather(ed_v, [didx])
            el = jnp.where(el > 0, el, 0.2 * el)
            ee = jnp.exp(el - c)
            ee_st[pl.ds(g * 16, 16)] = ee
            ees.append(ee)

        pltpu.make_async_copy(h_hbm.at[cid].at[src_v.at[j]], rows_in,
                              sem).wait()

        @pl.when(j >= 2)
        def _wr():
            pltpu.make_async_copy(rows_sc, acc_sp.at[dst_v.at[j]],
                                  sem_r).wait()

        for g in range(CH // 16):
            for l in range(16):
                ei = g * 16 + l
                w = ees[g][l]
                for d in range(DH // 16):
                    rows_sc[ei, pl.ds(d * 16, 16)] = (
                        rows_in[ei, pl.ds(d * 16, 16)] * w)
        pltpu.async_copy(rows_sc, acc_sp.at[dst_v.at[j]], sem_r, add=True)
        pltpu.async_copy(ee_st.at[pl.ds(0, CH)], den_sp.at[dst_v.at[j]],
                         sem_d, add=True)

    _gather(0, rows_in0)

    @pl.loop(0, NCHUNK, step=2)
    def _chunk(j):
        _gather(j + 1, rows_in1)
        _process(j, rows_in0, rows_sc0, ee_buf)

        @pl.when(j + 2 < NCHUNK)
        def _pf():
            _gather(j + 2, rows_in0)

        _process(j + 1, rows_in1, rows_sc1, ee_buf1)

    _scat_wait(NCHUNK - 2, rows_sc0, ee_buf)
    _scat_wait(NCHUNK - 1, rows_sc1, ee_buf1)

    plsc.subcore_barrier()

    def _stripe_out(pieces):
        off = 0
        for p in pieces:
            pltpu.sync_copy(acc_sp.at[pl.ds(base + off, p)],
                            zbuf.at[pl.ds(0, p)])
            pltpu.sync_copy(den_sp.at[pl.ds(base + off, p)],
                            den_v.at[pl.ds(0, p)])

            @pl.loop(0, p)
            def _nrm(r):
                ridx = jnp.broadcast_to(r, (16,)).astype(jnp.int32)
                w = 1.0 / (plsc.load_gather(den_v, [ridx]) + 1e-16)
                for g in range(DH // 16):
                    zbuf[r, pl.ds(g * 16, 16)] = zbuf[r, pl.ds(g * 16, 16)] * w

            pltpu.sync_copy(zbuf.at[pl.ds(0, p)],
                            acc_out.at[cid, pl.ds(base + off, p)])
            off += p

    @pl.when(sid < NS - 1)
    def _oa():
        _stripe_out(PIECES_A)

    @pl.when(sid == NS - 1)
    def _ob():
        _stripe_out(PIECES_B)


def _dots(h, a_s, a_d):
    es = jnp.sum(h * a_s, axis=1, keepdims=True)
    ed = jnp.sum(h * a_d, axis=1, keepdims=True)
    return es, ed


def _split_store(h_ref, h):
    h_ref[0] = h[:, :DH]
    h_ref[1] = h[:, DH:]


def _tc_in_body(x_ref, w_ref, as_ref, ad_ref, h_ref, es_ref, ed_ref):
    h = jnp.dot(x_ref[...], w_ref[...], preferred_element_type=jnp.float32)
    _split_store(h_ref, h)
    es, ed = _dots(h, as_ref[...], ad_ref[...])
    es_ref[...] = es
    ed_ref[...] = ed


def _combine(acc_ref, b_ref):
    a = acc_ref[...]
    return jnp.concatenate([a[0], a[1]], axis=1) + b_ref[...]


def _tc_mid_body(acc_ref, b_ref, w_ref, as_ref, ad_ref,
                 h_ref, es_ref, ed_ref):
    o = _combine(acc_ref, b_ref)
    o = jnp.where(o > 0, o, 0.01 * o)
    h = jnp.dot(o, w_ref[...], preferred_element_type=jnp.float32)
    _split_store(h_ref, h)
    es, ed = _dots(h, as_ref[...], ad_ref[...])
    es_ref[...] = es
    ed_ref[...] = ed


def _tc_out_body(acc_ref, b_ref, o_ref):
    o = _combine(acc_ref, b_ref)
    m = jnp.max(o, axis=1, keepdims=True)
    z = o - m
    o_ref[...] = z - jnp.log(jnp.sum(jnp.exp(z), axis=1, keepdims=True))


_bw = pl.BlockSpec((D, D), lambda i: (0, 0))
_bvec = pl.BlockSpec((1, D), lambda i: (0, 0))
_brows = pl.BlockSpec((ROWS_BLK, D), lambda i: (i, 0))
_bacc = pl.BlockSpec((NC, ROWS_BLK, DH), lambda i: (0, i, 0))
_be = pl.BlockSpec((ROWS_BLK, 1), lambda i: (i, 0))

_he_out = [
    jax.ShapeDtypeStruct((NC, N, DH), jnp.float32),
    jax.ShapeDtypeStruct((N, 1), jnp.float32),
    jax.ShapeDtypeStruct((N, 1), jnp.float32),
]

_tc_in = pl.pallas_call(
    _tc_in_body, grid=(GRID,),
    in_specs=[_brows, _bw, _bvec, _bvec],
    out_specs=[_bacc, _be, _be],
    out_shape=_he_out)

_tc_mid = pl.pallas_call(
    _tc_mid_body, grid=(GRID,),
    in_specs=[_bacc, _bvec, _bw, _bvec, _bvec],
    out_specs=[_bacc, _be, _be],
    out_shape=_he_out)

_tc_out = pl.pallas_call(
    _tc_out_body, grid=(GRID,),
    in_specs=[_bacc, _bvec],
    out_specs=_brows,
    out_shape=jax.ShapeDtypeStruct((N, D), jnp.float32))


def kernel(x, edge_index, W1, a1_src, a1_dst, b1, W2, a2_src, a2_dst, b2):
    src = edge_index[0].astype(jnp.int32).reshape(NS, NCHUNK, CH)
    dst = edge_index[1].astype(jnp.int32).reshape(NS, NCHUNK, CH)

    h1, es1, ed1 = _tc_in(x, W1, a1_src.reshape(1, D), a1_dst.reshape(1, D))
    acc1 = _sc_edge_layer(h1, src, dst, es1.reshape(N), ed1.reshape(N))
    h2, es2, ed2 = _tc_mid(acc1, b1.reshape(1, D), W2,
                           a2_src.reshape(1, D), a2_dst.reshape(1, D))
    acc2 = _sc_edge_layer(h2, src, dst, es2.reshape(N), ed2.reshape(N))
    return _tc_out(acc2, b2.reshape(1, D))


# speedup vs baseline: 1.0129x; 1.0129x over previous
import functools

import jax
import jax.numpy as jnp
from jax import lax
from jax.experimental import pallas as pl
from jax.experimental.pallas import tpu as pltpu
from jax.experimental.pallas import tpu_sc as plsc

N = 10000
E = 320000
D = 128
DH = D // 2

NC = 2
NS = 16
EPT = E // NS
CH = 80
NCHUNK = EPT // CH
STRIPE_A = 632
PIECES_A = (120, 120, 120, 120, 120, 32)
PIECES_B = (120, 120, 120, 120, 40)

ROWS_BLK = 1000
GRID = N // ROWS_BLK

_mesh = plsc.VectorSubcoreMesh(core_axis_name="c", subcore_axis_name="s")


@functools.partial(
    pl.kernel,
    out_type=jax.ShapeDtypeStruct((NC, N, DH), jnp.float32),
    mesh=_mesh,
    compiler_params=pltpu.CompilerParams(needs_layout_passes=False,
                                         use_tc_tiling_on_sc=False),
    scratch_types=[
        pltpu.VMEM((NCHUNK, CH), jnp.int32),
        pltpu.VMEM((NCHUNK, CH), jnp.int32),
        pltpu.VMEM((N,), jnp.float32),
        pltpu.VMEM((N,), jnp.float32),
        pltpu.VMEM((CH, DH), jnp.float32),
        pltpu.VMEM((CH, DH), jnp.float32),
        pltpu.VMEM((CH, DH), jnp.float32),
        pltpu.VMEM((CH, DH), jnp.float32),
        pltpu.VMEM((128,), jnp.float32),
        pltpu.VMEM((128,), jnp.float32),
        pltpu.VMEM((120, DH), jnp.float32),
        pltpu.VMEM((120,), jnp.float32),
        pltpu.VMEM_SHARED((N, DH), jnp.float32),
        pltpu.VMEM_SHARED((N,), jnp.float32),
        pltpu.SemaphoreType.DMA,
        pltpu.SemaphoreType.DMA,
        pltpu.SemaphoreType.DMA,
    ],
)
def _sc_edge_layer(h_hbm, src_hbm, dst_hbm, es_hbm, ed_hbm, acc_out,
                   src_v, dst_v, es_v, ed_v, rows_in0, rows_in1, rows_sc0,
                   rows_sc1, ee_buf, ee_buf1, zbuf, den_v, acc_sp, den_sp,
                   sem, sem_r, sem_d):
    cid = lax.axis_index("c")
    sid = lax.axis_index("s")
    base = sid * STRIPE_A

    @pl.loop(0, 120)
    def _zb(i):
        for g in range(DH // 16):
            zbuf[i, pl.ds(g * 16, 16)] = jnp.zeros((16,), jnp.float32)

    def _stripe_zero(pieces):
        off = 0
        for p in pieces:
            pltpu.sync_copy(zbuf.at[pl.ds(0, p)],
                            acc_sp.at[pl.ds(base + off, p)])
            off += p

    @pl.when(sid < NS - 1)
    def _za():
        _stripe_zero(PIECES_A)

    @pl.when(sid == NS - 1)
    def _zbb():
        _stripe_zero(PIECES_B)

    @pl.when(sid == 0)
    def _zden():
        @pl.loop(0, N // 128)
        def _z(i):
            for g in range(8):
                es_v[pl.ds(i * 128 + g * 16, 16)] = jnp.zeros(
                    (16,), jnp.float32)
        pltpu.sync_copy(es_v, den_sp)

    pltpu.sync_copy(src_hbm.at[sid], src_v)
    pltpu.sync_copy(dst_hbm.at[sid], dst_v)
    pltpu.sync_copy(es_hbm, es_v)
    pltpu.sync_copy(ed_hbm, ed_v)

    def _vmax(ref):
        @pl.loop(0, N // 128, init_carry=jnp.full((16,), -jnp.inf, jnp.float32))
        def m(i, carry):
            for g in range(8):
                carry = jnp.maximum(carry, ref[pl.ds(i * 128 + g * 16, 16)])
            return carry
        out = m
        for sh in (8, 4, 2, 1):
            ee_buf[pl.ds(0, 16)] = out
            idx = lax.iota(jnp.int32, 16) ^ sh
            out = jnp.maximum(out, plsc.load_gather(ee_buf, [idx]))
        return out

    c0 = _vmax(es_v) + _vmax(ed_v)
    c = jnp.where(c0 > 0, c0, 0.2 * c0)

    plsc.subcore_barrier()

    def _gather(j, buf):
        pltpu.async_copy(h_hbm.at[cid].at[src_v.at[j]], buf, sem)

    def _scat_wait(j, rows_sc, ee):
        pltpu.make_async_copy(rows_sc, acc_sp.at[dst_v.at[j]], sem_r).wait()
        pltpu.make_async_copy(ee.at[pl.ds(0, CH)], den_sp.at[dst_v.at[j]],
                              sem_d).wait()

    def _process(j, rows_in, rows_sc, ee_st):
        @pl.when(j >= 2)
        def _wd():
            pltpu.make_async_copy(ee_st.at[pl.ds(0, CH)],
                                  den_sp.at[dst_v.at[j]], sem_d).wait()

        ees = []
        for g in range(CH // 16):
            sidx = src_v[j, pl.ds(g * 16, 16)]
            didx = dst_v[j, pl.ds(g * 16, 16)]
            el = plsc.load_gather(es_v, [sidx]) + plsc.load_gather(ed_v, [didx])
            el = jnp.where(el > 0, el, 0.2 * el)
            ee = jnp.exp(el - c)
            ee_st[pl.ds(g * 16, 16)] = ee
            ees.append(ee)

        pltpu.make_async_copy(h_hbm.at[cid].at[src_v.at[j]], rows_in,
                              sem).wait()

        @pl.when(j >= 2)
        def _wr():
            pltpu.make_async_copy(rows_sc, acc_sp.at[dst_v.at[j]],
                                  sem_r).wait()

        for g in range(CH // 16):
            for l in range(16):
                ei = g * 16 + l
                w = ees[g][l]
                for d in range(DH // 16):
                    rows_sc[ei, pl.ds(d * 16, 16)] = (
                        rows_in[ei, pl.ds(d * 16, 16)] * w)
        pltpu.async_copy(rows_sc, acc_sp.at[dst_v.at[j]], sem_r, add=True)
        pltpu.async_copy(ee_st.at[pl.ds(0, CH)], den_sp.at[dst_v.at[j]],
                         sem_d, add=True)

    _gather(0, rows_in0)

    @pl.loop(0, NCHUNK, step=2)
    def _chunk(j):
        _gather(j + 1, rows_in1)
        _process(j, rows_in0, rows_sc0, ee_buf)

        @pl.when(j + 2 < NCHUNK)
        def _pf():
            _gather(j + 2, rows_in0)

        _process(j + 1, rows_in1, rows_sc1, ee_buf1)

    _scat_wait(NCHUNK - 2, rows_sc0, ee_buf)
    _scat_wait(NCHUNK - 1, rows_sc1, ee_buf1)

    plsc.subcore_barrier()

    def _stripe_out(pieces):
        off = 0
        for p in pieces:
            pltpu.sync_copy(acc_sp.at[pl.ds(base + off, p)],
                            zbuf.at[pl.ds(0, p)])
            pltpu.sync_copy(den_sp.at[pl.ds(base + off, p)],
                            den_v.at[pl.ds(0, p)])

            @pl.loop(0, p)
            def _nrm(r):
                ridx = jnp.broadcast_to(r, (16,)).astype(jnp.int32)
                w = 1.0 / (plsc.load_gather(den_v, [ridx]) + 1e-16)
                for g in range(DH // 16):
                    zbuf[r, pl.ds(g * 16, 16)] = zbuf[r, pl.ds(g * 16, 16)] * w

            pltpu.sync_copy(zbuf.at[pl.ds(0, p)],
                            acc_out.at[cid, pl.ds(base + off, p)])
            off += p

    @pl.when(sid < NS - 1)
    def _oa():
        _stripe_out(PIECES_A)

    @pl.when(sid == NS - 1)
    def _ob():
        _stripe_out(PIECES_B)


def _dots(h, a_s, a_d):
    es = jnp.sum(h * a_s, axis=1, keepdims=True)
    ed = jnp.sum(h * a_d, axis=1, keepdims=True)
    return (jnp.broadcast_to(es, (es.shape[0], 8)),
            jnp.broadcast_to(ed, (ed.shape[0], 8)))


def _split_store(h_ref, h):
    h_ref[0] = h[:, :DH]
    h_ref[1] = h[:, DH:]


def _tc_in_body(x_ref, w_ref, as_ref, ad_ref, h_ref, es_ref, ed_ref):
    h = jnp.dot(x_ref[...], w_ref[...], preferred_element_type=jnp.float32)
    _split_store(h_ref, h)
    es, ed = _dots(h, as_ref[...], ad_ref[...])
    es_ref[...] = es
    ed_ref[...] = ed


def _combine(acc_ref, b_ref):
    a = acc_ref[...]
    return jnp.concatenate([a[0], a[1]], axis=1) + b_ref[...]


def _tc_mid_body(acc_ref, b_ref, w_ref, as_ref, ad_ref,
                 h_ref, es_ref, ed_ref):
    o = _combine(acc_ref, b_ref)
    o = jnp.where(o > 0, o, 0.01 * o)
    h = jnp.dot(o, w_ref[...], preferred_element_type=jnp.float32)
    _split_store(h_ref, h)
    es, ed = _dots(h, as_ref[...], ad_ref[...])
    es_ref[...] = es
    ed_ref[...] = ed


def _tc_out_body(acc_ref, b_ref, o_ref):
    o = _combine(acc_ref, b_ref)
    m = jnp.max(o, axis=1, keepdims=True)
    z = o - m
    o_ref[...] = z - jnp.log(jnp.sum(jnp.exp(z), axis=1, keepdims=True))


_bw = pl.BlockSpec((D, D), lambda i: (0, 0))
_bvec = pl.BlockSpec((1, D), lambda i: (0, 0))
_brows = pl.BlockSpec((ROWS_BLK, D), lambda i: (i, 0))
_bacc = pl.BlockSpec((NC, ROWS_BLK, DH), lambda i: (0, i, 0))
_be = pl.BlockSpec((ROWS_BLK, 8), lambda i: (i, 0))

_he_out = [
    jax.ShapeDtypeStruct((NC, N, DH), jnp.float32),
    jax.ShapeDtypeStruct((N, 8), jnp.float32),
    jax.ShapeDtypeStruct((N, 8), jnp.float32),
]

_tc_in = pl.pallas_call(
    _tc_in_body, grid=(GRID,),
    in_specs=[_brows, _bw, _bvec, _bvec],
    out_specs=[_bacc, _be, _be],
    out_shape=_he_out)

_tc_mid = pl.pallas_call(
    _tc_mid_body, grid=(GRID,),
    in_specs=[_bacc, _bvec, _bw, _bvec, _bvec],
    out_specs=[_bacc, _be, _be],
    out_shape=_he_out)

_tc_out = pl.pallas_call(
    _tc_out_body, grid=(GRID,),
    in_specs=[_bacc, _bvec],
    out_specs=_brows,
    out_shape=jax.ShapeDtypeStruct((N, D), jnp.float32))


def kernel(x, edge_index, W1, a1_src, a1_dst, b1, W2, a2_src, a2_dst, b2):
    src = edge_index[0].astype(jnp.int32).reshape(NS, NCHUNK, CH)
    dst = edge_index[1].astype(jnp.int32).reshape(NS, NCHUNK, CH)

    h1, es1, ed1 = _tc_in(x, W1, a1_src.reshape(1, D), a1_dst.reshape(1, D))
    acc1 = _sc_edge_layer(h1, src, dst, es1[:, 0], ed1[:, 0])
    h2, es2, ed2 = _tc_mid(acc1, b1.reshape(1, D), W2,
                           a2_src.reshape(1, D), a2_dst.reshape(1, D))
    acc2 = _sc_edge_layer(h2, src, dst, es2[:, 0], ed2[:, 0])
    return _tc_out(acc2, b2.reshape(1, D))
